# baseline (speedup 1.0000x reference)
import jax
import jax.numpy as jnp
from jax import lax
from jax.experimental import pallas as pl
from jax.experimental.pallas import tpu as pltpu
from jax.experimental.pallas import tpu_sc as plsc

_VOCAB = 1000000
_D = 64
_PADW = 72
_BATCH = 16384
_HIST = 20
_N = _BATCH * _HIST

_NC = 2
_NS = 16
_NW = _NC * _NS
_BPW = _BATCH // _NW
_PER_W = _N // _NW
_BPC = 8
_CHUNK = _BPC * _HIST
_NCHUNK = _PER_W // _CHUNK

_mesh = plsc.VectorSubcoreMesh(
    core_axis_name="c", subcore_axis_name="s",
    num_cores=_NC, num_subcores=_NS,
)


def _body(idx_hbm, tab_hbm, out_hbm, idx_v, gbufs, gsems, osems):
    wid = lax.axis_index("s") * _NC + lax.axis_index("c")
    r0 = wid * _PER_W
    pltpu.sync_copy(idx_hbm.at[pl.ds(r0, _PER_W)], idx_v)

    def fire_gather(c, p):
        for half in range(2):
            pltpu.async_copy(
                tab_hbm.at[idx_v.at[pl.ds(c * _CHUNK + half * 80, 80)]],
                gbufs.at[p, pl.ds(half * 80, 80)], gsems.at[p])

    def drain_gather(p):
        pltpu.make_async_copy(tab_hbm.at[pl.ds(0, _CHUNK)], gbufs.at[p],
                              gsems.at[p]).wait()

    def fire_out(c, p):
        for k in range(_BPC):
            pltpu.async_copy(
                gbufs.at[p, pl.ds(k * _HIST, _HIST), pl.ds(0, _D)],
                out_hbm.at[wid * _BPW + c * _BPC + k],
                osems.at[p])

    def wait_out(p):
        for _ in range(_BPC):
            pltpu.make_async_copy(
                gbufs.at[p, pl.ds(0, _HIST), pl.ds(0, _D)],
                out_hbm.at[0],
                osems.at[p]).wait()

    fire_gather(0, 0)

    def step(s, carry):
        for p in range(2):
            c = 2 * s + p

            @pl.when(c + 1 < _NCHUNK)
            def _():
                fire_gather(c + 1, 1 - p)

            drain_gather(p)

            @pl.when(c >= 2)
            def _():
                wait_out(p)

            fire_out(c, p)
        return carry

    lax.fori_loop(0, _NCHUNK // 2, step, 0)
    wait_out(0)
    wait_out(1)


_gather = pl.kernel(
    _body,
    out_type=jax.ShapeDtypeStruct((_BATCH, _HIST, _D), jnp.float32),
    mesh=_mesh,
    scratch_types=[
        pltpu.VMEM((_PER_W,), jnp.int32),
        pltpu.VMEM((2, _CHUNK, _PADW), jnp.float32),
        pltpu.SemaphoreType.DMA((2,)),
        pltpu.SemaphoreType.DMA((2,)),
    ],
    compiler_params=pltpu.CompilerParams(use_tc_tiling_on_sc=False),
)


def kernel(inputs, w, b):
    idx = inputs.astype(jnp.int32).reshape(_N)
    w2 = jnp.pad(w, ((0, 0), (0, _PADW - _D)))
    return _gather(idx, w2)


# speedup vs baseline: 1.6146x; 1.6146x over previous
import jax
import jax.numpy as jnp
from jax import lax
from jax.experimental import pallas as pl
from jax.experimental.pallas import tpu as pltpu
from jax.experimental.pallas import tpu_sc as plsc

_VOCAB = 1000000
_D = 64
_PADW = 128
_BATCH = 16384
_HIST = 20
_N = _BATCH * _HIST

_NC = 2
_NS = 16
_NW = _NC * _NS
_BPW = _BATCH // _NW
_PER_W = _N // _NW
_BPC = 8
_CHUNK = _BPC * _HIST
_NCHUNK = _PER_W // _CHUNK

_mesh = plsc.VectorSubcoreMesh(
    core_axis_name="c", subcore_axis_name="s",
    num_cores=_NC, num_subcores=_NS,
)


def _body(idx_hbm, tab_hbm, out_hbm, idx_v, gbufs, gsems, osems):
    wid = lax.axis_index("s") * _NC + lax.axis_index("c")
    r0 = wid * _PER_W
    pltpu.sync_copy(idx_hbm.at[pl.ds(r0, _PER_W)], idx_v)

    def fire_gather(c, p):
        for half in range(2):
            pltpu.async_copy(
                tab_hbm.at[idx_v.at[pl.ds(c * _CHUNK + half * 80, 80)]],
                gbufs.at[p, pl.ds(half * 80, 80)], gsems.at[p])

    def drain_gather(p):
        pltpu.make_async_copy(tab_hbm.at[pl.ds(0, _CHUNK)], gbufs.at[p],
                              gsems.at[p]).wait()

    def fire_out(c, p):
        for k in range(_BPC):
            pltpu.async_copy(
                gbufs.at[p, pl.ds(k * _HIST, _HIST), pl.ds(0, _D)],
                out_hbm.at[wid * _BPW + c * _BPC + k],
                osems.at[p])

    def wait_out(p):
        for _ in range(_BPC):
            pltpu.make_async_copy(
                gbufs.at[p, pl.ds(0, _HIST), pl.ds(0, _D)],
                out_hbm.at[0],
                osems.at[p]).wait()

    fire_gather(0, 0)

    def step(s, carry):
        for p in range(2):
            c = 2 * s + p

            @pl.when(c + 1 < _NCHUNK)
            def _():
                fire_gather(c + 1, 1 - p)

            drain_gather(p)

            @pl.when(c >= 2)
            def _():
                wait_out(p)

            fire_out(c, p)
        return carry

    lax.fori_loop(0, _NCHUNK // 2, step, 0)
    wait_out(0)
    wait_out(1)


_gather = pl.kernel(
    _body,
    out_type=jax.ShapeDtypeStruct((_BATCH, _HIST, _D), jnp.float32),
    mesh=_mesh,
    scratch_types=[
        pltpu.VMEM((_PER_W,), jnp.int32),
        pltpu.VMEM((2, _CHUNK, _PADW), jnp.float32),
        pltpu.SemaphoreType.DMA((2,)),
        pltpu.SemaphoreType.DMA((2,)),
    ],
    compiler_params=pltpu.CompilerParams(use_tc_tiling_on_sc=False),
)


def kernel(inputs, w, b):
    idx = inputs.astype(jnp.int32).reshape(_N)
    w2 = jnp.pad(w, ((0, 0), (0, _PADW - _D)))
    return _gather(idx, w2)
